# Initial kernel scaffold; baseline (speedup 1.0000x reference)
import jax
import jax.numpy as jnp
from jax.experimental import pallas as pl


def kernel(predictions, labels, scores):
    raise NotImplementedError("write your pallas kernel here")


# speedup vs baseline: 10.6622x; 10.6622x over previous
import functools

import jax
import jax.numpy as jnp
from jax import lax
from jax.experimental import pallas as pl
from jax.experimental.pallas import tpu as pltpu
from jax.experimental.pallas import tpu_sc as plsc

L = 128
V = 16
NCH = L // V

def _bf(s):
    return lax.broadcast_in_dim(jnp.float32(s), (V,), ())


def _bi(s):
    return lax.broadcast_in_dim(jnp.int32(s), (V,), ())


def _sc_body(lx1, ly1, lx2, ly2, px1, py1, px2, py2, out,
             lx1v, ly1v, lx2v, ly2v, px1v, py1v, px2v, py2v,
             jm, areap, rowmax, argcol, seen, perm, outv):
    _IOTA = lax.iota(jnp.int32, V)
    _LANE0 = _IOTA == 0

    @pl.when((lax.axis_index("c") == 0) & (lax.axis_index("s") == 0))
    def _():
        for src, dst in ((lx1, lx1v), (ly1, ly1v), (lx2, lx2v), (ly2, ly2v),
                         (px1, px1v), (py1, py1v), (px2, px2v), (py2, py2v)):
            pltpu.sync_copy(src, dst)

        neg1f = jnp.full((V,), -1.0, jnp.float32)
        neg1i = jnp.full((V,), -1, jnp.int32)
        zerof = jnp.full((V,), 0.0, jnp.float32)
        zeroi = jnp.full((V,), 0, jnp.int32)
        onei = jnp.full((V,), 1, jnp.int32)

        for j in range(NCH):
            s = pl.ds(j * V, V)
            areap[s] = (px2v[s] - px1v[s]) * (py2v[s] - py1v[s])
            seen[s] = zeroi
            perm[s] = _IOTA + j * V

        def row_scan(l, use_seen):
            base = l * L
            vms = []
            vmax = neg1f
            for j in range(NCH):
                v = jm[pl.ds(base + j * V, V)]
                if use_seen:
                    v = jnp.where(seen[pl.ds(j * V, V)] == zeroi, v, neg1f)
                vms.append(v)
                vmax = jnp.maximum(vmax, v)
            m = jnp.max(vmax)
            mb = _bf(m)
            amax = neg1i
            for j in range(NCH):
                amax = jnp.maximum(
                    amax, jnp.where(vms[j] == mb, _IOTA + j * V, neg1i))
            return m, jnp.max(amax)

        def set1(ref, i, vec):
            plsc.store_scatter(ref, [_bi(i)], vec, mask=_LANE0)

        def p1_body(l, carry):
            lb = _bi(l)
            x1 = plsc.load_gather(lx1v, [lb])
            y1 = plsc.load_gather(ly1v, [lb])
            x2 = plsc.load_gather(lx2v, [lb])
            y2 = plsc.load_gather(ly2v, [lb])
            al = (x2 - x1) * (y2 - y1)
            base = l * L
            vms = []
            vmax = neg1f
            for j in range(NCH):
                s = pl.ds(j * V, V)
                ix1 = jnp.maximum(x1, px1v[s])
                iy1 = jnp.maximum(y1, py1v[s])
                ix2 = jnp.minimum(x2, px2v[s])
                iy2 = jnp.minimum(y2, py2v[s])
                inter = jnp.maximum(ix2 - ix1, zerof) * jnp.maximum(
                    iy2 - iy1, zerof)
                v = inter / (al + areap[s] - inter)
                jm[pl.ds(base + j * V, V)] = v
                vms.append(v)
                vmax = jnp.maximum(vmax, v)
            m = jnp.max(vmax)
            mb = _bf(m)
            amax = neg1i
            for j in range(NCH):
                amax = jnp.maximum(
                    amax, jnp.where(vms[j] == mb, _IOTA + j * V, neg1i))
            set1(rowmax, l, mb)
            set1(argcol, l, _bi(jnp.max(amax)))
            return carry

        lax.fori_loop(0, L, p1_body, 0)

        def step(k, carry):
            vs = []
            vmax = neg1f
            for j in range(NCH):
                s = pl.ds(j * V, V)
                v = jnp.where(seen[s] == zeroi, rowmax[s], neg1f)
                vs.append(v)
                vmax = jnp.maximum(vmax, v)
            m = jnp.max(vmax)
            mb = _bf(m)
            amax = neg1i
            for j in range(NCH):
                amax = jnp.maximum(
                    amax, jnp.where(vs[j] == mb, _IOTA + j * V, neg1i))
            r = jnp.max(amax)
            c = jnp.max(plsc.load_gather(argcol, [_bi(r)]))

            rb, cb = r * L, c * L
            for j in range(NCH):
                a = jm[pl.ds(rb + j * V, V)]
                b = jm[pl.ds(cb + j * V, V)]
                jm[pl.ds(rb + j * V, V)] = b
                jm[pl.ds(cb + j * V, V)] = a
            for ref in (rowmax, argcol, perm):
                a = plsc.load_gather(ref, [_bi(r)])
                b = plsc.load_gather(ref, [_bi(c)])
                plsc.store_scatter(ref, [_bi(r)], b, mask=_LANE0)
                plsc.store_scatter(ref, [_bi(c)], a, mask=_LANE0)
            set1(seen, c, onei)

            cv = _bi(c)
            for j in range(NCH):
                def find(_):
                    s = pl.ds(j * V, V)
                    hit = (argcol[s] == cv) & (seen[s] == zeroi)
                    return jnp.max(jnp.where(hit, _IOTA + j * V, neg1i))

                def rec_body(i):
                    m2, a2 = row_scan(i, use_seen=True)
                    set1(rowmax, i, _bf(m2))
                    set1(argcol, i, _bi(a2))
                    return find(0)

                lax.while_loop(lambda i: i >= 0, rec_body, find(0))
            return carry

        lax.fori_loop(0, L, step, 0)

        acc = zerof
        for j in range(NCH):
            s = pl.ds(j * V, V)
            idx = perm[s]
            for lv, pv in ((lx1v, px1v), (ly1v, py1v),
                           (lx2v, px2v), (ly2v, py2v)):
                acc = acc + jnp.abs(plsc.load_gather(lv, [idx]) - pv[s])
        outv[...] = _bf(jnp.sum(acc) * (1.0 / (4 * L)))
        pltpu.sync_copy(outv, out)


@jax.jit
def _run(cols):
    mesh = plsc.VectorSubcoreMesh(core_axis_name="c", subcore_axis_name="s")
    f = pl.kernel(
        _sc_body,
        out_type=jax.ShapeDtypeStruct((V,), jnp.float32),
        mesh=mesh,
        scratch_types=(
            [pltpu.VMEM((L,), jnp.float32) for _ in range(8)]
            + [pltpu.VMEM((L * L,), jnp.float32),
               pltpu.VMEM((L,), jnp.float32),
               pltpu.VMEM((L,), jnp.float32),
               pltpu.VMEM((L,), jnp.int32),
               pltpu.VMEM((L,), jnp.int32),
               pltpu.VMEM((L,), jnp.int32),
               pltpu.VMEM((V,), jnp.float32)]),
        compiler_params=pltpu.CompilerParams(needs_layout_passes=False),
    )
    return f(*cols)[0]


def kernel(predictions, labels, scores):
    del scores
    cols = tuple(labels[:, i] for i in range(4)) + tuple(
        predictions[:, i] for i in range(4))
    return _run(cols)


# speedup vs baseline: 11.5997x; 1.0879x over previous
import functools

import jax
import jax.numpy as jnp
from jax import lax
from jax.experimental import pallas as pl
from jax.experimental.pallas import tpu as pltpu
from jax.experimental.pallas import tpu_sc as plsc

L = 128
V = 16
NCH = L // V

def _bf(s):
    return lax.broadcast_in_dim(jnp.float32(s), (V,), ())


def _bi(s):
    return lax.broadcast_in_dim(jnp.int32(s), (V,), ())


def _sc_body(lx1, ly1, lx2, ly2, px1, py1, px2, py2, out,
             lx1v, ly1v, lx2v, ly2v, px1v, py1v, px2v, py2v,
             jm, areap, rowmax, argcol, seen, perm, outv):
    _IOTA = lax.iota(jnp.int32, V)
    _LANE0 = _IOTA == 0

    @pl.when((lax.axis_index("c") == 0) & (lax.axis_index("s") == 0))
    def _():
        for src, dst in ((lx1, lx1v), (ly1, ly1v), (lx2, lx2v), (ly2, ly2v),
                         (px1, px1v), (py1, py1v), (px2, px2v), (py2, py2v)):
            pltpu.sync_copy(src, dst)

        neg1f = jnp.full((V,), -1.0, jnp.float32)
        neg1i = jnp.full((V,), -1, jnp.int32)
        zerof = jnp.full((V,), 0.0, jnp.float32)
        zeroi = jnp.full((V,), 0, jnp.int32)
        onei = jnp.full((V,), 1, jnp.int32)

        for j in range(NCH):
            s = pl.ds(j * V, V)
            areap[s] = (px2v[s] - px1v[s]) * (py2v[s] - py1v[s])
            seen[s] = zeroi
            perm[s] = _IOTA + j * V

        def row_scan(ri):
            base = ri * L
            vms = []
            vmax = neg1f
            for j in range(NCH):
                v = jnp.where(seen[pl.ds(j * V, V)] == zeroi,
                              jm[pl.ds(base + j * V, V)], neg1f)
                vms.append(v)
                vmax = jnp.maximum(vmax, v)
            m = jnp.max(vmax)
            mb = _bf(m)
            amax = neg1i
            for j in range(NCH):
                amax = jnp.maximum(
                    amax, jnp.where(vms[j] == mb, _IOTA + j * V, neg1i))
            return m, jnp.max(amax)

        def set1(ref, i, vec):
            plsc.store_scatter(ref, [_bi(i)], vec, mask=_LANE0)

        def sread(ref, i):
            return jnp.max(plsc.load_gather(ref, [_bi(i)]))

        def p1_body(l, carry):
            lb = _bi(l)
            x1 = plsc.load_gather(lx1v, [lb])
            y1 = plsc.load_gather(ly1v, [lb])
            x2 = plsc.load_gather(lx2v, [lb])
            y2 = plsc.load_gather(ly2v, [lb])
            al = (x2 - x1) * (y2 - y1)
            base = l * L
            vms = []
            vmax = neg1f
            for j in range(NCH):
                s = pl.ds(j * V, V)
                ix1 = jnp.maximum(x1, px1v[s])
                iy1 = jnp.maximum(y1, py1v[s])
                ix2 = jnp.minimum(x2, px2v[s])
                iy2 = jnp.minimum(y2, py2v[s])
                inter = jnp.maximum(ix2 - ix1, zerof) * jnp.maximum(
                    iy2 - iy1, zerof)
                v = inter / (al + areap[s] - inter)
                jm[pl.ds(base + j * V, V)] = v
                vms.append(v)
                vmax = jnp.maximum(vmax, v)
            m = jnp.max(vmax)
            mb = _bf(m)
            amax = neg1i
            for j in range(NCH):
                amax = jnp.maximum(
                    amax, jnp.where(vms[j] == mb, _IOTA + j * V, neg1i))
            set1(rowmax, l, mb)
            set1(argcol, l, _bi(jnp.max(amax)))
            return carry

        lax.fori_loop(0, L, p1_body, 0)

        def step(k, carry):
            vs = []
            vmax = neg1f
            for j in range(NCH):
                v = rowmax[pl.ds(j * V, V)]
                vs.append(v)
                vmax = jnp.maximum(vmax, v)
            m = jnp.max(vmax)
            mb = _bf(m)
            amax = neg1i
            for j in range(NCH):
                amax = jnp.maximum(
                    amax, jnp.where(vs[j] == mb, _IOTA + j * V, neg1i))
            r = jnp.max(amax)
            c = sread(argcol, r)

            for ref in (rowmax, argcol, perm):
                a = plsc.load_gather(ref, [_bi(r)])
                b = plsc.load_gather(ref, [_bi(c)])
                plsc.store_scatter(ref, [_bi(r)], b, mask=_LANE0)
                plsc.store_scatter(ref, [_bi(c)], a, mask=_LANE0)
            set1(seen, c, onei)
            set1(rowmax, c, neg1f)
            set1(argcol, c, neg1i)

            cv = _bi(c)

            def find():
                hmax = neg1i
                for j in range(NCH):
                    hit = argcol[pl.ds(j * V, V)] == cv
                    hmax = jnp.maximum(
                        hmax, jnp.where(hit, _IOTA + j * V, neg1i))
                return jnp.max(hmax)

            def rec_body(i):
                m2, a2 = row_scan(sread(perm, i))
                set1(rowmax, i, _bf(m2))
                set1(argcol, i, _bi(a2))
                return find()

            lax.while_loop(lambda i: i >= 0, rec_body, find())
            return carry

        lax.fori_loop(0, L, step, 0)

        acc = zerof
        for j in range(NCH):
            s = pl.ds(j * V, V)
            idx = perm[s]
            for lv, pv in ((lx1v, px1v), (ly1v, py1v),
                           (lx2v, px2v), (ly2v, py2v)):
                acc = acc + jnp.abs(plsc.load_gather(lv, [idx]) - pv[s])
        outv[...] = _bf(jnp.sum(acc) * (1.0 / (4 * L)))
        pltpu.sync_copy(outv, out)


@jax.jit
def _run(cols):
    mesh = plsc.VectorSubcoreMesh(core_axis_name="c", subcore_axis_name="s")
    f = pl.kernel(
        _sc_body,
        out_type=jax.ShapeDtypeStruct((V,), jnp.float32),
        mesh=mesh,
        scratch_types=(
            [pltpu.VMEM((L,), jnp.float32) for _ in range(8)]
            + [pltpu.VMEM((L * L,), jnp.float32),
               pltpu.VMEM((L,), jnp.float32),
               pltpu.VMEM((L,), jnp.float32),
               pltpu.VMEM((L,), jnp.int32),
               pltpu.VMEM((L,), jnp.int32),
               pltpu.VMEM((L,), jnp.int32),
               pltpu.VMEM((V,), jnp.float32)]),
        compiler_params=pltpu.CompilerParams(needs_layout_passes=False),
    )
    return f(*cols)[0]


def kernel(predictions, labels, scores):
    del scores
    cols = tuple(labels[:, i] for i in range(4)) + tuple(
        predictions[:, i] for i in range(4))
    return _run(cols)


# speedup vs baseline: 11.8422x; 1.0209x over previous
import functools

import jax
import jax.numpy as jnp
from jax import lax
from jax.experimental import pallas as pl
from jax.experimental.pallas import tpu as pltpu
from jax.experimental.pallas import tpu_sc as plsc

L = 128
V = 16
NCH = L // V

def _bf(s):
    return lax.broadcast_in_dim(jnp.float32(s), (V,), ())


def _bi(s):
    return lax.broadcast_in_dim(jnp.int32(s), (V,), ())


def _sc_body(lab, pred, out,
             labv, predv,
             lx1v, ly1v, lx2v, ly2v, px1v, py1v, px2v, py2v,
             jm, areap, rowmax, argcol, seen, perm, outv):
    _IOTA = lax.iota(jnp.int32, V)
    _LANE0 = _IOTA == 0

    @pl.when((lax.axis_index("c") == 0) & (lax.axis_index("s") == 0))
    def _():
        pltpu.sync_copy(lab, labv)
        pltpu.sync_copy(pred, predv)

        for t, (lcol, pcol) in enumerate(
                ((lx1v, px1v), (ly1v, py1v), (lx2v, px2v), (ly2v, py2v))):
            for j in range(NCH):
                s = pl.ds(j * V, V)
                idx = (_IOTA + j * V) * 4 + t
                lcol[s] = plsc.load_gather(labv, [idx])
                pcol[s] = plsc.load_gather(predv, [idx])

        neg1f = jnp.full((V,), -1.0, jnp.float32)
        neg1i = jnp.full((V,), -1, jnp.int32)
        zerof = jnp.full((V,), 0.0, jnp.float32)
        zeroi = jnp.full((V,), 0, jnp.int32)
        onei = jnp.full((V,), 1, jnp.int32)

        for j in range(NCH):
            s = pl.ds(j * V, V)
            areap[s] = (px2v[s] - px1v[s]) * (py2v[s] - py1v[s])
            seen[s] = zeroi
            perm[s] = _IOTA + j * V

        def row_scan(ri):
            base = ri * L
            vms = []
            vmax = neg1f
            for j in range(NCH):
                v = jnp.where(seen[pl.ds(j * V, V)] == zeroi,
                              jm[pl.ds(base + j * V, V)], neg1f)
                vms.append(v)
                vmax = jnp.maximum(vmax, v)
            m = jnp.max(vmax)
            mb = _bf(m)
            amax = neg1i
            for j in range(NCH):
                amax = jnp.maximum(
                    amax, jnp.where(vms[j] == mb, _IOTA + j * V, neg1i))
            return m, jnp.max(amax)

        def set1(ref, i, vec):
            plsc.store_scatter(ref, [_bi(i)], vec, mask=_LANE0)

        def sread(ref, i):
            return jnp.max(plsc.load_gather(ref, [_bi(i)]))

        def p1_body(l, carry):
            lb = _bi(l)
            x1 = plsc.load_gather(lx1v, [lb])
            y1 = plsc.load_gather(ly1v, [lb])
            x2 = plsc.load_gather(lx2v, [lb])
            y2 = plsc.load_gather(ly2v, [lb])
            al = (x2 - x1) * (y2 - y1)
            base = l * L
            vms = []
            vmax = neg1f
            for j in range(NCH):
                s = pl.ds(j * V, V)
                ix1 = jnp.maximum(x1, px1v[s])
                iy1 = jnp.maximum(y1, py1v[s])
                ix2 = jnp.minimum(x2, px2v[s])
                iy2 = jnp.minimum(y2, py2v[s])
                inter = jnp.maximum(ix2 - ix1, zerof) * jnp.maximum(
                    iy2 - iy1, zerof)
                v = inter / (al + areap[s] - inter)
                jm[pl.ds(base + j * V, V)] = v
                vms.append(v)
                vmax = jnp.maximum(vmax, v)
            m = jnp.max(vmax)
            mb = _bf(m)
            amax = neg1i
            for j in range(NCH):
                amax = jnp.maximum(
                    amax, jnp.where(vms[j] == mb, _IOTA + j * V, neg1i))
            set1(rowmax, l, mb)
            set1(argcol, l, _bi(jnp.max(amax)))
            return carry

        lax.fori_loop(0, L, p1_body, 0)

        def step(k, carry):
            vs = []
            vmax = neg1f
            for j in range(NCH):
                v = rowmax[pl.ds(j * V, V)]
                vs.append(v)
                vmax = jnp.maximum(vmax, v)
            m = jnp.max(vmax)
            mb = _bf(m)
            amax = neg1i
            for j in range(NCH):
                amax = jnp.maximum(
                    amax, jnp.where(vs[j] == mb, _IOTA + j * V, neg1i))
            r = jnp.max(amax)
            c = sread(argcol, r)

            for ref in (rowmax, argcol, perm):
                a = plsc.load_gather(ref, [_bi(r)])
                b = plsc.load_gather(ref, [_bi(c)])
                plsc.store_scatter(ref, [_bi(r)], b, mask=_LANE0)
                plsc.store_scatter(ref, [_bi(c)], a, mask=_LANE0)
            set1(seen, c, onei)
            set1(rowmax, c, neg1f)
            set1(argcol, c, neg1i)

            cv = _bi(c)

            def find():
                hmax = neg1i
                for j in range(NCH):
                    hit = argcol[pl.ds(j * V, V)] == cv
                    hmax = jnp.maximum(
                        hmax, jnp.where(hit, _IOTA + j * V, neg1i))
                return jnp.max(hmax)

            def rec_body(i):
                m2, a2 = row_scan(sread(perm, i))
                set1(rowmax, i, _bf(m2))
                set1(argcol, i, _bi(a2))
                return find()

            lax.while_loop(lambda i: i >= 0, rec_body, find())
            return carry

        lax.fori_loop(0, L, step, 0)

        acc = zerof
        for j in range(NCH):
            s = pl.ds(j * V, V)
            idx = perm[s]
            for lv, pv in ((lx1v, px1v), (ly1v, py1v),
                           (lx2v, px2v), (ly2v, py2v)):
                acc = acc + jnp.abs(plsc.load_gather(lv, [idx]) - pv[s])
        outv[...] = _bf(jnp.sum(acc) * (1.0 / (4 * L)))
        pltpu.sync_copy(outv, out)


@jax.jit
def _run(cols):
    mesh = plsc.VectorSubcoreMesh(core_axis_name="c", subcore_axis_name="s")
    f = pl.kernel(
        _sc_body,
        out_type=jax.ShapeDtypeStruct((V,), jnp.float32),
        mesh=mesh,
        scratch_types=(
            [pltpu.VMEM((4 * L,), jnp.float32) for _ in range(2)]
            + [pltpu.VMEM((L,), jnp.float32) for _ in range(8)]
            + [pltpu.VMEM((L * L,), jnp.float32),
               pltpu.VMEM((L,), jnp.float32),
               pltpu.VMEM((L,), jnp.float32),
               pltpu.VMEM((L,), jnp.int32),
               pltpu.VMEM((L,), jnp.int32),
               pltpu.VMEM((L,), jnp.int32),
               pltpu.VMEM((V,), jnp.float32)]),
        compiler_params=pltpu.CompilerParams(needs_layout_passes=False),
    )
    return f(*cols)[0]


def kernel(predictions, labels, scores):
    del scores
    return _run((labels.reshape(-1), predictions.reshape(-1)))


# speedup vs baseline: 13.8525x; 1.1698x over previous
import jax
import jax.numpy as jnp
from jax import lax
from jax.experimental import pallas as pl
from jax.experimental.pallas import tpu as pltpu
from jax.experimental.pallas import tpu_sc as plsc

L = 128
V = 16
NCH = L // V


def _bi(s):
    return lax.broadcast_in_dim(jnp.int32(s), (V,), ())


def _sc_body(lab, pred, out,
             labv, predv,
             lx1v, ly1v, lx2v, ly2v, px1v, py1v, px2v, py2v,
             jm, areap, rowmax, argcol, colmask, perm, outv):
    iota = lax.iota(jnp.int32, V)
    lane0 = iota == 0
    lane15 = iota == V - 1
    splat15 = jnp.full((V,), V - 1, jnp.int32)

    def splat_last(v):
        return jax.lax.gather(
            v, splat15[:, None],
            jax.lax.GatherDimensionNumbers(
                offset_dims=(), collapsed_slice_dims=(0,),
                start_index_map=(0,)),
            (1,), mode=jax.lax.GatherScatterMode.PROMISE_IN_BOUNDS)

    @pl.when((lax.axis_index("c") == 0) & (lax.axis_index("s") == 0))
    def _():
        pltpu.sync_copy(lab, labv)
        pltpu.sync_copy(pred, predv)

        neg1f = jnp.full((V,), -1.0, jnp.float32)
        neg1i = jnp.full((V,), -1, jnp.int32)
        neg4f = jnp.full((V,), -4.0, jnp.float32)
        zerof = jnp.full((V,), 0.0, jnp.float32)

        for t, (lcol, pcol) in enumerate(
                ((lx1v, px1v), (ly1v, py1v), (lx2v, px2v), (ly2v, py2v))):
            for j in range(NCH):
                s = pl.ds(j * V, V)
                idx = (iota + j * V) * 4 + t
                lcol[s] = plsc.load_gather(labv, [idx])
                pcol[s] = plsc.load_gather(predv, [idx])
        for j in range(NCH):
            s = pl.ds(j * V, V)
            areap[s] = (px2v[s] - px1v[s]) * (py2v[s] - py1v[s])
            colmask[s] = zerof
            perm[s] = iota + j * V

        def p1_body(l, carry):
            lb = _bi(l)
            x1 = plsc.load_gather(lx1v, [lb])
            y1 = plsc.load_gather(ly1v, [lb])
            x2 = plsc.load_gather(lx2v, [lb])
            y2 = plsc.load_gather(ly2v, [lb])
            al = (x2 - x1) * (y2 - y1)
            base = l * L
            vms = []
            vmax = neg1f
            for j in range(NCH):
                s = pl.ds(j * V, V)
                ix1 = jnp.maximum(x1, px1v[s])
                iy1 = jnp.maximum(y1, py1v[s])
                ix2 = jnp.minimum(x2, px2v[s])
                iy2 = jnp.minimum(y2, py2v[s])
                inter = jnp.maximum(ix2 - ix1, zerof) * jnp.maximum(
                    iy2 - iy1, zerof)
                v = inter / (al + areap[s] - inter)
                jm[pl.ds(base + j * V, V)] = v
                vms.append(v)
                vmax = jnp.maximum(vmax, v)
            cm = plsc.cummax(vmax)
            plsc.store_scatter(rowmax, [lb], cm, mask=lane15)
            mspl = splat_last(cm)
            amax = neg1i
            for j in range(NCH):
                amax = jnp.maximum(
                    amax, jnp.where(vms[j] == mspl, iota + j * V, neg1i))
            plsc.store_scatter(argcol, [lb], plsc.cummax(amax), mask=lane15)
            return carry

        lax.fori_loop(0, L, p1_body, 0)

        def row_scan(riv, iv):
            base = riv * L
            vms = []
            vmax = neg4f
            for j in range(NCH):
                v = plsc.load_gather(jm, [base + (iota + j * V)])
                v = v + colmask[pl.ds(j * V, V)]
                vms.append(v)
                vmax = jnp.maximum(vmax, v)
            cm = plsc.cummax(vmax)
            plsc.store_scatter(rowmax, [iv], cm, mask=lane15)
            mspl = splat_last(cm)
            amax = neg1i
            for j in range(NCH):
                amax = jnp.maximum(
                    amax, jnp.where(vms[j] == mspl, iota + j * V, neg1i))
            plsc.store_scatter(argcol, [iv], plsc.cummax(amax), mask=lane15)

        def step(k, carry):
            vs = []
            vmax = neg1f
            for j in range(NCH):
                v = rowmax[pl.ds(j * V, V)]
                vs.append(v)
                vmax = jnp.maximum(vmax, v)
            mspl = splat_last(plsc.cummax(vmax))
            amax = neg1i
            for j in range(NCH):
                amax = jnp.maximum(
                    amax, jnp.where(vs[j] == mspl, iota + j * V, neg1i))
            rv = splat_last(plsc.cummax(amax))
            cv = plsc.load_gather(argcol, [rv])

            for ref in (rowmax, argcol, perm):
                a = plsc.load_gather(ref, [rv])
                b = plsc.load_gather(ref, [cv])
                plsc.store_scatter(ref, [rv], b, mask=lane0)
                plsc.store_scatter(ref, [cv], a, mask=lane0)
            plsc.store_scatter(rowmax, [cv], neg1f, mask=lane0)
            plsc.store_scatter(argcol, [cv], neg1i, mask=lane0)
            plsc.store_scatter(colmask, [cv], neg4f, mask=lane0)

            def find():
                hmax = neg1i
                for j in range(NCH):
                    hit = argcol[pl.ds(j * V, V)] == cv
                    hmax = jnp.maximum(
                        hmax, jnp.where(hit, iota + j * V, neg1i))
                return jnp.max(hmax)

            def rec_body(i):
                iv = _bi(i)
                row_scan(plsc.load_gather(perm, [iv]), iv)
                return find()

            lax.while_loop(lambda i: i >= 0, rec_body, find())
            return carry

        lax.fori_loop(0, L, step, 0)

        acc = zerof
        for j in range(NCH):
            s = pl.ds(j * V, V)
            idx = perm[s]
            for lv, pv in ((lx1v, px1v), (ly1v, py1v),
                           (lx2v, px2v), (ly2v, py2v)):
                acc = acc + jnp.abs(plsc.load_gather(lv, [idx]) - pv[s])
        outv[...] = lax.broadcast_in_dim(
            jnp.sum(acc) * jnp.float32(1.0 / (4 * L)), (V,), ())
        pltpu.sync_copy(outv, out)


@jax.jit
def _run(cols):
    mesh = plsc.VectorSubcoreMesh(core_axis_name="c", subcore_axis_name="s")
    f = pl.kernel(
        _sc_body,
        out_type=jax.ShapeDtypeStruct((V,), jnp.float32),
        mesh=mesh,
        scratch_types=(
            [pltpu.VMEM((4 * L,), jnp.float32) for _ in range(2)]
            + [pltpu.VMEM((L,), jnp.float32) for _ in range(8)]
            + [pltpu.VMEM((L * L,), jnp.float32),
               pltpu.VMEM((L,), jnp.float32),
               pltpu.VMEM((L,), jnp.float32),
               pltpu.VMEM((L,), jnp.int32),
               pltpu.VMEM((L,), jnp.float32),
               pltpu.VMEM((L,), jnp.int32),
               pltpu.VMEM((V,), jnp.float32)]),
        compiler_params=pltpu.CompilerParams(needs_layout_passes=False),
    )
    return f(*cols)[0]


def kernel(predictions, labels, scores):
    del scores
    return _run((labels.reshape(-1), predictions.reshape(-1)))


# speedup vs baseline: 16.8710x; 1.2179x over previous
import jax
import jax.numpy as jnp
from jax import lax
from jax.experimental import pallas as pl
from jax.experimental.pallas import tpu as pltpu
from jax.experimental.pallas import tpu_sc as plsc

L = 128
V = 16
NCH = L // V
RPT = L // V


def _bi(s):
    return lax.broadcast_in_dim(jnp.int32(s), (V,), ())


def _sc_body(lab, pred, out,
             labv, predv,
             lx1v, ly1v, lx2v, ly2v, px1v, py1v, px2v, py2v,
             jm8, rm16, ac16,
             jm_sh, rm_sh, ac_sh,
             jm, areap, rowmax, argcol, colmask, perm, tmpf, tmpi, outv):
    iota = lax.iota(jnp.int32, V)
    lane0 = iota == 0
    lane15 = iota == V - 1
    splat15 = jnp.full((V,), V - 1, jnp.int32)

    def splat_last(v):
        return jax.lax.gather(
            v, splat15[:, None],
            jax.lax.GatherDimensionNumbers(
                offset_dims=(), collapsed_slice_dims=(0,),
                start_index_map=(0,)),
            (1,), mode=jax.lax.GatherScatterMode.PROMISE_IN_BOUNDS)

    neg1f = jnp.full((V,), -1.0, jnp.float32)
    neg1i = jnp.full((V,), -1, jnp.int32)
    neg4f = jnp.full((V,), -4.0, jnp.float32)
    zerof = jnp.full((V,), 0.0, jnp.float32)

    sid = lax.axis_index("s")

    @pl.when(lax.axis_index("c") == 0)
    def _():
        pltpu.sync_copy(lab, labv)
        pltpu.sync_copy(pred, predv)
        for t, (lcol, pcol) in enumerate(
                ((lx1v, px1v), (ly1v, py1v), (lx2v, px2v), (ly2v, py2v))):
            for j in range(NCH):
                s = pl.ds(j * V, V)
                idx = (iota + j * V) * 4 + t
                lcol[s] = plsc.load_gather(labv, [idx])
                pcol[s] = plsc.load_gather(predv, [idx])
        for j in range(NCH):
            s = pl.ds(j * V, V)
            areap[s] = (px2v[s] - px1v[s]) * (py2v[s] - py1v[s])

        for q in range(RPT):
            lb = _bi(sid * RPT + q)
            x1 = plsc.load_gather(lx1v, [lb])
            y1 = plsc.load_gather(ly1v, [lb])
            x2 = plsc.load_gather(lx2v, [lb])
            y2 = plsc.load_gather(ly2v, [lb])
            al = (x2 - x1) * (y2 - y1)
            vms = []
            vmax = neg1f
            for j in range(NCH):
                s = pl.ds(j * V, V)
                ix1 = jnp.maximum(x1, px1v[s])
                iy1 = jnp.maximum(y1, py1v[s])
                ix2 = jnp.minimum(x2, px2v[s])
                iy2 = jnp.minimum(y2, py2v[s])
                inter = jnp.maximum(ix2 - ix1, zerof) * jnp.maximum(
                    iy2 - iy1, zerof)
                v = inter / (al + areap[s] - inter)
                jm8[pl.ds(q * L + j * V, V)] = v
                vms.append(v)
                vmax = jnp.maximum(vmax, v)
            cm = plsc.cummax(vmax)
            plsc.store_scatter(rm16, [_bi(q)], cm, mask=lane15)
            mspl = splat_last(cm)
            amax = neg1i
            for j in range(NCH):
                amax = jnp.maximum(
                    amax, jnp.where(vms[j] == mspl, iota + j * V, neg1i))
            plsc.store_scatter(ac16, [_bi(q)], plsc.cummax(amax), mask=lane15)

        pltpu.sync_copy(jm8, jm_sh.at[pl.ds(sid * (RPT * L), RPT * L)])
        pltpu.sync_copy(rm16, rm_sh.at[pl.ds(sid * V, V)])
        pltpu.sync_copy(ac16, ac_sh.at[pl.ds(sid * V, V)])
        plsc.subcore_barrier()

        @pl.when(sid == 0)
        def _():
            pltpu.sync_copy(jm_sh, jm)
            pltpu.sync_copy(rm_sh, tmpf)
            pltpu.sync_copy(ac_sh, tmpi)
            for j in range(NCH):
                i = iota + j * V
                cidx = (i // RPT) * V + (i % RPT)
                rowmax[pl.ds(j * V, V)] = plsc.load_gather(tmpf, [cidx])
                argcol[pl.ds(j * V, V)] = plsc.load_gather(tmpi, [cidx])
                colmask[pl.ds(j * V, V)] = zerof
                perm[pl.ds(j * V, V)] = i

            def row_scan(riv, iv):
                base = riv * L
                vms = []
                vmax = neg4f
                for j in range(NCH):
                    v = plsc.load_gather(jm, [base + (iota + j * V)])
                    v = v + colmask[pl.ds(j * V, V)]
                    vms.append(v)
                    vmax = jnp.maximum(vmax, v)
                cm = plsc.cummax(vmax)
                plsc.store_scatter(rowmax, [iv], cm, mask=lane15)
                mspl = splat_last(cm)
                amax = neg1i
                for j in range(NCH):
                    amax = jnp.maximum(
                        amax, jnp.where(vms[j] == mspl, iota + j * V, neg1i))
                plsc.store_scatter(argcol, [iv], plsc.cummax(amax),
                                   mask=lane15)

            def step(k, carry):
                vs = []
                vmax = neg1f
                for j in range(NCH):
                    v = rowmax[pl.ds(j * V, V)]
                    vs.append(v)
                    vmax = jnp.maximum(vmax, v)
                mspl = splat_last(plsc.cummax(vmax))
                amax = neg1i
                for j in range(NCH):
                    amax = jnp.maximum(
                        amax, jnp.where(vs[j] == mspl, iota + j * V, neg1i))
                rv = splat_last(plsc.cummax(amax))
                cv = plsc.load_gather(argcol, [rv])

                for ref in (rowmax, argcol, perm):
                    a = plsc.load_gather(ref, [rv])
                    b = plsc.load_gather(ref, [cv])
                    plsc.store_scatter(ref, [rv], b, mask=lane0)
                    plsc.store_scatter(ref, [cv], a, mask=lane0)
                plsc.store_scatter(rowmax, [cv], neg1f, mask=lane0)
                plsc.store_scatter(argcol, [cv], neg1i, mask=lane0)
                plsc.store_scatter(colmask, [cv], neg4f, mask=lane0)

                def find():
                    hmax = neg1i
                    for j in range(NCH):
                        hit = argcol[pl.ds(j * V, V)] == cv
                        hmax = jnp.maximum(
                            hmax, jnp.where(hit, iota + j * V, neg1i))
                    return jnp.max(hmax)

                def rec_body(i):
                    iv = _bi(i)
                    row_scan(plsc.load_gather(perm, [iv]), iv)
                    return find()

                lax.while_loop(lambda i: i >= 0, rec_body, find())
                return carry

            lax.fori_loop(0, L, step, 0)

            acc = zerof
            for j in range(NCH):
                s = pl.ds(j * V, V)
                idx = perm[s]
                for lv, pv in ((lx1v, px1v), (ly1v, py1v),
                               (lx2v, px2v), (ly2v, py2v)):
                    acc = acc + jnp.abs(plsc.load_gather(lv, [idx]) - pv[s])
            outv[...] = lax.broadcast_in_dim(
                jnp.sum(acc) * jnp.float32(1.0 / (4 * L)), (V,), ())
            pltpu.sync_copy(outv, out)


@jax.jit
def _run(cols):
    mesh = plsc.VectorSubcoreMesh(core_axis_name="c", subcore_axis_name="s")
    f = pl.kernel(
        _sc_body,
        out_type=jax.ShapeDtypeStruct((V,), jnp.float32),
        mesh=mesh,
        scratch_types=(
            [pltpu.VMEM((4 * L,), jnp.float32) for _ in range(2)]
            + [pltpu.VMEM((L,), jnp.float32) for _ in range(8)]
            + [pltpu.VMEM((RPT * L,), jnp.float32),
               pltpu.VMEM((V,), jnp.float32),
               pltpu.VMEM((V,), jnp.int32),
               pltpu.VMEM_SHARED((L * L,), jnp.float32),
               pltpu.VMEM_SHARED((V * V,), jnp.float32),
               pltpu.VMEM_SHARED((V * V,), jnp.int32),
               pltpu.VMEM((L * L,), jnp.float32),
               pltpu.VMEM((L,), jnp.float32),
               pltpu.VMEM((L,), jnp.float32),
               pltpu.VMEM((L,), jnp.int32),
               pltpu.VMEM((L,), jnp.float32),
               pltpu.VMEM((L,), jnp.int32),
               pltpu.VMEM((V * V,), jnp.float32),
               pltpu.VMEM((V * V,), jnp.int32),
               pltpu.VMEM((V,), jnp.float32)]),
        compiler_params=pltpu.CompilerParams(needs_layout_passes=False),
    )
    return f(*cols)[0]


def kernel(predictions, labels, scores):
    del scores
    return _run((labels.reshape(-1), predictions.reshape(-1)))


# speedup vs baseline: 19.2167x; 1.1390x over previous
import jax
import jax.numpy as jnp
from jax import lax
from jax.experimental import pallas as pl
from jax.experimental.pallas import tpu as pltpu
from jax.experimental.pallas import tpu_sc as plsc

L = 128
V = 16
NCH = L // V
RPT = L // V


def _bi(s):
    return lax.broadcast_in_dim(jnp.int32(s), (V,), ())


def _sc_body(lab, pred, out,
             labv, predv,
             lx1v, ly1v, lx2v, ly2v, px1v, py1v, px2v, py2v,
             jm8, rm16, ac16,
             jm_sh, rm_sh, ac_sh,
             jm, areap, rowmax, argcol, colmask, perm, tmpf, tmpi, outv):
    iota = lax.iota(jnp.int32, V)
    lane0 = iota == 0
    lane15 = iota == V - 1
    splat15 = jnp.full((V,), V - 1, jnp.int32)

    def splat_last(v):
        return jax.lax.gather(
            v, splat15[:, None],
            jax.lax.GatherDimensionNumbers(
                offset_dims=(), collapsed_slice_dims=(0,),
                start_index_map=(0,)),
            (1,), mode=jax.lax.GatherScatterMode.PROMISE_IN_BOUNDS)

    neg1f = jnp.full((V,), -1.0, jnp.float32)
    neg1i = jnp.full((V,), -1, jnp.int32)
    neg4f = jnp.full((V,), -4.0, jnp.float32)
    zerof = jnp.full((V,), 0.0, jnp.float32)

    sid = lax.axis_index("s")

    @pl.when(lax.axis_index("c") == 0)
    def _():
        pltpu.sync_copy(lab, labv)
        pltpu.sync_copy(pred, predv)
        for t, (lcol, pcol) in enumerate(
                ((lx1v, px1v), (ly1v, py1v), (lx2v, px2v), (ly2v, py2v))):
            for j in range(NCH):
                s = pl.ds(j * V, V)
                idx = (iota + j * V) * 4 + t
                lcol[s] = plsc.load_gather(labv, [idx])
                pcol[s] = plsc.load_gather(predv, [idx])
        for j in range(NCH):
            s = pl.ds(j * V, V)
            areap[s] = (px2v[s] - px1v[s]) * (py2v[s] - py1v[s])

        for q in range(RPT):
            lb = _bi(sid * RPT + q)
            x1 = plsc.load_gather(lx1v, [lb])
            y1 = plsc.load_gather(ly1v, [lb])
            x2 = plsc.load_gather(lx2v, [lb])
            y2 = plsc.load_gather(ly2v, [lb])
            al = (x2 - x1) * (y2 - y1)
            vms = []
            vmax = neg1f
            for j in range(NCH):
                s = pl.ds(j * V, V)
                ix1 = jnp.maximum(x1, px1v[s])
                iy1 = jnp.maximum(y1, py1v[s])
                ix2 = jnp.minimum(x2, px2v[s])
                iy2 = jnp.minimum(y2, py2v[s])
                inter = jnp.maximum(ix2 - ix1, zerof) * jnp.maximum(
                    iy2 - iy1, zerof)
                v = inter / (al + areap[s] - inter)
                jm8[pl.ds(q * L + j * V, V)] = v
                vms.append(v)
                vmax = jnp.maximum(vmax, v)
            cm = plsc.cummax(vmax)
            plsc.store_scatter(rm16, [_bi(q)], cm, mask=lane15)
            mspl = splat_last(cm)
            amax = neg1i
            for j in range(NCH):
                amax = jnp.maximum(
                    amax, jnp.where(vms[j] == mspl, iota + j * V, neg1i))
            plsc.store_scatter(ac16, [_bi(q)], plsc.cummax(amax), mask=lane15)

        pltpu.sync_copy(jm8, jm_sh.at[pl.ds(sid * (RPT * L), RPT * L)])
        pltpu.sync_copy(rm16, rm_sh.at[pl.ds(sid * V, V)])
        pltpu.sync_copy(ac16, ac_sh.at[pl.ds(sid * V, V)])
        plsc.subcore_barrier()

        @pl.when(sid == 0)
        def _():
            pltpu.sync_copy(jm_sh, jm)
            pltpu.sync_copy(rm_sh, tmpf)
            pltpu.sync_copy(ac_sh, tmpi)
            for j in range(NCH):
                i = iota + j * V
                cidx = (i // RPT) * V + (i % RPT)
                rowmax[pl.ds(j * V, V)] = plsc.load_gather(tmpf, [cidx])
                argcol[pl.ds(j * V, V)] = plsc.load_gather(tmpi, [cidx])
                colmask[pl.ds(j * V, V)] = zerof
                perm[pl.ds(j * V, V)] = i

            def row_scan(riv, iv):
                base = riv * L
                vms = []
                vmax = neg4f
                for j in range(NCH):
                    v = plsc.load_gather(jm, [base + (iota + j * V)])
                    v = v + colmask[pl.ds(j * V, V)]
                    vms.append(v)
                    vmax = jnp.maximum(vmax, v)
                cm = plsc.cummax(vmax)
                plsc.store_scatter(rowmax, [iv], cm, mask=lane15)
                mspl = splat_last(cm)
                amax = neg1i
                for j in range(NCH):
                    amax = jnp.maximum(
                        amax, jnp.where(vms[j] == mspl, iota + j * V, neg1i))
                plsc.store_scatter(argcol, [iv], plsc.cummax(amax),
                                   mask=lane15)

            lane01 = iota < 2
            def step(k, carry):
                acs, vs = [], []
                vmax = neg1f
                for j in range(NCH):
                    acs.append(argcol[pl.ds(j * V, V)])
                    v = rowmax[pl.ds(j * V, V)]
                    vs.append(v)
                    vmax = jnp.maximum(vmax, v)
                mspl = splat_last(plsc.cummax(vmax))
                amax = neg1i
                for j in range(NCH):
                    amax = jnp.maximum(
                        amax, jnp.where(vs[j] == mspl, iota + j * V, neg1i))
                rv = splat_last(plsc.cummax(amax))
                cv = plsc.load_gather(argcol, [rv])

                idx_rc = jnp.where(lane0, rv, cv)
                idx_cr = jnp.where(lane0, cv, rv)
                diag = rv == cv
                pswp = plsc.load_gather(perm, [idx_cr])
                bmax = plsc.load_gather(rowmax, [cv])
                bac = plsc.load_gather(argcol, [cv])
                plsc.store_scatter(perm, [idx_rc], pswp, mask=lane01)
                plsc.store_scatter(
                    rowmax, [idx_rc],
                    jnp.where(lane0 & ~diag, bmax, neg1f), mask=lane01)
                nac = jnp.where(lane0 & ~diag, bac, neg1i)
                plsc.store_scatter(argcol, [idx_rc], nac, mask=lane01)
                plsc.store_scatter(colmask, [cv], neg4f, mask=lane0)

                bacp = jnp.where(diag, neg1i, bac)
                cands = []
                hmax = neg1i
                for j in range(NCH):
                    lanevec = iota + j * V
                    pac = jnp.where(lanevec == rv, bacp,
                                    jnp.where(lanevec == cv, neg1i, acs[j]))
                    cand = jnp.where(pac == cv, lanevec, neg1i)
                    cands.append(cand)
                    hmax = jnp.maximum(hmax, cand)

                def rec_body(carry2):
                    i = carry2[0]
                    cnds = carry2[1:]
                    iv = _bi(i)
                    row_scan(plsc.load_gather(perm, [iv]), iv)
                    nmax = neg1i
                    out_c = []
                    for cnd in cnds:
                        cnd = jnp.where(cnd == iv, neg1i, cnd)
                        out_c.append(cnd)
                        nmax = jnp.maximum(nmax, cnd)
                    return (jnp.max(nmax),) + tuple(out_c)

                lax.while_loop(lambda cr: cr[0] >= 0, rec_body,
                               (jnp.max(hmax),) + tuple(cands))
                return carry

            lax.fori_loop(0, L, step, 0)

            acc = zerof
            for j in range(NCH):
                s = pl.ds(j * V, V)
                idx = perm[s]
                for lv, pv in ((lx1v, px1v), (ly1v, py1v),
                               (lx2v, px2v), (ly2v, py2v)):
                    acc = acc + jnp.abs(plsc.load_gather(lv, [idx]) - pv[s])
            outv[...] = lax.broadcast_in_dim(
                jnp.sum(acc) * jnp.float32(1.0 / (4 * L)), (V,), ())
            pltpu.sync_copy(outv, out)


@jax.jit
def _run(cols):
    mesh = plsc.VectorSubcoreMesh(core_axis_name="c", subcore_axis_name="s")
    f = pl.kernel(
        _sc_body,
        out_type=jax.ShapeDtypeStruct((V,), jnp.float32),
        mesh=mesh,
        scratch_types=(
            [pltpu.VMEM((4 * L,), jnp.float32) for _ in range(2)]
            + [pltpu.VMEM((L,), jnp.float32) for _ in range(8)]
            + [pltpu.VMEM((RPT * L,), jnp.float32),
               pltpu.VMEM((V,), jnp.float32),
               pltpu.VMEM((V,), jnp.int32),
               pltpu.VMEM_SHARED((L * L,), jnp.float32),
               pltpu.VMEM_SHARED((V * V,), jnp.float32),
               pltpu.VMEM_SHARED((V * V,), jnp.int32),
               pltpu.VMEM((L * L,), jnp.float32),
               pltpu.VMEM((L,), jnp.float32),
               pltpu.VMEM((L,), jnp.float32),
               pltpu.VMEM((L,), jnp.int32),
               pltpu.VMEM((L,), jnp.float32),
               pltpu.VMEM((L,), jnp.int32),
               pltpu.VMEM((V * V,), jnp.float32),
               pltpu.VMEM((V * V,), jnp.int32),
               pltpu.VMEM((V,), jnp.float32)]),
        compiler_params=pltpu.CompilerParams(needs_layout_passes=False),
    )
    return f(*cols)[0]


def kernel(predictions, labels, scores):
    del scores
    return _run((labels.reshape(-1), predictions.reshape(-1)))


# speedup vs baseline: 22.0680x; 1.1484x over previous
import jax
import jax.numpy as jnp
from jax import lax
from jax.experimental import pallas as pl
from jax.experimental.pallas import tpu as pltpu
from jax.experimental.pallas import tpu_sc as plsc

L = 128
V = 16
NCH = L // V
RPT = L // V


def _bi(s):
    return lax.broadcast_in_dim(jnp.int32(s), (V,), ())


def _sc_body(lab, pred, out,
             labv, predv,
             lx1v, ly1v, lx2v, ly2v, px1v, py1v, px2v, py2v,
             jm8, rm16, ac16,
             jm_sh, rm_sh, ac_sh,
             jm, areap, rowmax, argcol, colmask, perm, tmpf, tmpi, outv):
    iota = lax.iota(jnp.int32, V)
    lane0 = iota == 0
    lane15 = iota == V - 1
    splat15 = jnp.full((V,), V - 1, jnp.int32)

    def splat_last(v):
        return jax.lax.gather(
            v, splat15[:, None],
            jax.lax.GatherDimensionNumbers(
                offset_dims=(), collapsed_slice_dims=(0,),
                start_index_map=(0,)),
            (1,), mode=jax.lax.GatherScatterMode.PROMISE_IN_BOUNDS)

    neg1f = jnp.full((V,), -1.0, jnp.float32)
    neg1i = jnp.full((V,), -1, jnp.int32)
    neg4f = jnp.full((V,), -4.0, jnp.float32)
    zerof = jnp.full((V,), 0.0, jnp.float32)

    sid = lax.axis_index("s")

    @pl.when(lax.axis_index("c") == 0)
    def _():
        pltpu.sync_copy(lab, labv)
        pltpu.sync_copy(pred, predv)
        for t, (lcol, pcol) in enumerate(
                ((lx1v, px1v), (ly1v, py1v), (lx2v, px2v), (ly2v, py2v))):
            for j in range(NCH):
                s = pl.ds(j * V, V)
                idx = (iota + j * V) * 4 + t
                lcol[s] = plsc.load_gather(labv, [idx])
                pcol[s] = plsc.load_gather(predv, [idx])
        for j in range(NCH):
            s = pl.ds(j * V, V)
            areap[s] = (px2v[s] - px1v[s]) * (py2v[s] - py1v[s])

        def max_lastarg(vms):
            lanemax = vms[0]
            for j in range(1, NCH):
                lanemax = jnp.maximum(lanemax, vms[j])
            lc = jnp.zeros((V,), jnp.int32)
            for j in range(1, NCH):
                lc = jnp.where(vms[j] == lanemax,
                               jnp.full((V,), j, jnp.int32), lc)
            cm = plsc.cummax(lanemax)
            mspl = splat_last(cm)
            flat = jnp.where(lanemax == mspl, lc * V + iota, neg1i)
            return cm, plsc.cummax(flat)

        for q in range(RPT):
            lb = _bi(sid * RPT + q)
            x1 = plsc.load_gather(lx1v, [lb])
            y1 = plsc.load_gather(ly1v, [lb])
            x2 = plsc.load_gather(lx2v, [lb])
            y2 = plsc.load_gather(ly2v, [lb])
            al = (x2 - x1) * (y2 - y1)
            vms = []
            for j in range(NCH):
                s = pl.ds(j * V, V)
                ix1 = jnp.maximum(x1, px1v[s])
                iy1 = jnp.maximum(y1, py1v[s])
                ix2 = jnp.minimum(x2, px2v[s])
                iy2 = jnp.minimum(y2, py2v[s])
                inter = jnp.maximum(ix2 - ix1, zerof) * jnp.maximum(
                    iy2 - iy1, zerof)
                v = inter / (al + areap[s] - inter)
                jm8[pl.ds(q * L + j * V, V)] = v
                vms.append(v)
            cm, cam = max_lastarg(vms)
            plsc.store_scatter(rm16, [_bi(q)], cm, mask=lane15)
            plsc.store_scatter(ac16, [_bi(q)], cam, mask=lane15)

        pltpu.sync_copy(jm8, jm_sh.at[pl.ds(sid * (RPT * L), RPT * L)])
        pltpu.sync_copy(rm16, rm_sh.at[pl.ds(sid * V, V)])
        pltpu.sync_copy(ac16, ac_sh.at[pl.ds(sid * V, V)])
        plsc.subcore_barrier()

        @pl.when(sid == 0)
        def _():
            pltpu.sync_copy(jm_sh, jm)
            pltpu.sync_copy(rm_sh, tmpf)
            pltpu.sync_copy(ac_sh, tmpi)
            for j in range(NCH):
                i = iota + j * V
                cidx = (i // RPT) * V + (i % RPT)
                rowmax[pl.ds(j * V, V)] = plsc.load_gather(tmpf, [cidx])
                argcol[pl.ds(j * V, V)] = plsc.load_gather(tmpi, [cidx])
                colmask[pl.ds(j * V, V)] = zerof
                perm[pl.ds(j * V, V)] = i

            def row_scan(riv, iv):
                base = riv * L
                vms = []
                for j in range(NCH):
                    v = plsc.load_gather(jm, [base + (iota + j * V)])
                    vms.append(v + colmask[pl.ds(j * V, V)])
                cm, cam = max_lastarg(vms)
                plsc.store_scatter(rowmax, [iv], cm, mask=lane15)
                plsc.store_scatter(argcol, [iv], cam, mask=lane15)

            lane01 = iota < 2
            def step(k, carry):
                vs = [rowmax[pl.ds(j * V, V)] for j in range(NCH)]
                _, cam = max_lastarg(vs)
                rv = splat_last(cam)
                cv = plsc.load_gather(argcol, [rv])
                acs = [argcol[pl.ds(j * V, V)] for j in range(NCH)]

                idx_rc = jnp.where(lane0, rv, cv)
                idx_cr = jnp.where(lane0, cv, rv)
                diag = rv == cv
                pswp = plsc.load_gather(perm, [idx_cr])
                bmax = plsc.load_gather(rowmax, [cv])
                bac = plsc.load_gather(argcol, [cv])
                plsc.store_scatter(perm, [idx_rc], pswp, mask=lane01)
                plsc.store_scatter(
                    rowmax, [idx_rc],
                    jnp.where(lane0 & ~diag, bmax, neg1f), mask=lane01)
                nac = jnp.where(lane0 & ~diag, bac, neg1i)
                plsc.store_scatter(argcol, [idx_rc], nac, mask=lane01)
                plsc.store_scatter(colmask, [cv], neg4f, mask=lane0)

                bacp = jnp.where(diag, neg1i, bac)
                cands = []
                hmax = neg1i
                for j in range(NCH):
                    lanevec = iota + j * V
                    pac = jnp.where(lanevec == rv, bacp,
                                    jnp.where(lanevec == cv, neg1i, acs[j]))
                    cand = jnp.where(pac == cv, lanevec, neg1i)
                    cands.append(cand)
                    hmax = jnp.maximum(hmax, cand)

                def rec_body(carry2):
                    i = carry2[0]
                    cnds = carry2[1:]
                    iv = _bi(i)
                    row_scan(plsc.load_gather(perm, [iv]), iv)
                    nmax = neg1i
                    out_c = []
                    for cnd in cnds:
                        cnd = jnp.where(cnd == iv, neg1i, cnd)
                        out_c.append(cnd)
                        nmax = jnp.maximum(nmax, cnd)
                    return (jnp.max(nmax),) + tuple(out_c)

                lax.while_loop(lambda cr: cr[0] >= 0, rec_body,
                               (jnp.max(hmax),) + tuple(cands))
                return carry

            lax.fori_loop(0, L, step, 0)

            acc = zerof
            for j in range(NCH):
                s = pl.ds(j * V, V)
                idx = perm[s]
                for lv, pv in ((lx1v, px1v), (ly1v, py1v),
                               (lx2v, px2v), (ly2v, py2v)):
                    acc = acc + jnp.abs(plsc.load_gather(lv, [idx]) - pv[s])
            outv[...] = lax.broadcast_in_dim(
                jnp.sum(acc) * jnp.float32(1.0 / (4 * L)), (V,), ())
            pltpu.sync_copy(outv, out)


@jax.jit
def _run(cols):
    mesh = plsc.VectorSubcoreMesh(core_axis_name="c", subcore_axis_name="s")
    f = pl.kernel(
        _sc_body,
        out_type=jax.ShapeDtypeStruct((V,), jnp.float32),
        mesh=mesh,
        scratch_types=(
            [pltpu.VMEM((4 * L,), jnp.float32) for _ in range(2)]
            + [pltpu.VMEM((L,), jnp.float32) for _ in range(8)]
            + [pltpu.VMEM((RPT * L,), jnp.float32),
               pltpu.VMEM((V,), jnp.float32),
               pltpu.VMEM((V,), jnp.int32),
               pltpu.VMEM_SHARED((L * L,), jnp.float32),
               pltpu.VMEM_SHARED((V * V,), jnp.float32),
               pltpu.VMEM_SHARED((V * V,), jnp.int32),
               pltpu.VMEM((L * L,), jnp.float32),
               pltpu.VMEM((L,), jnp.float32),
               pltpu.VMEM((L,), jnp.float32),
               pltpu.VMEM((L,), jnp.int32),
               pltpu.VMEM((L,), jnp.float32),
               pltpu.VMEM((L,), jnp.int32),
               pltpu.VMEM((V * V,), jnp.float32),
               pltpu.VMEM((V * V,), jnp.int32),
               pltpu.VMEM((V,), jnp.float32)]),
        compiler_params=pltpu.CompilerParams(needs_layout_passes=False),
    )
    return f(*cols)[0]


def kernel(predictions, labels, scores):
    del scores
    return _run((labels.reshape(-1), predictions.reshape(-1)))


# speedup vs baseline: 22.3876x; 1.0145x over previous
import jax
import jax.numpy as jnp
from jax import lax
from jax.experimental import pallas as pl
from jax.experimental.pallas import tpu as pltpu
from jax.experimental.pallas import tpu_sc as plsc

L = 128
V = 16
NCH = L // V
RPT = L // V


def _bi(s):
    return lax.broadcast_in_dim(jnp.int32(s), (V,), ())


def _sc_body(lab, pred, out,
             labv, predv,
             lx1v, ly1v, lx2v, ly2v, px1v, py1v, px2v, py2v,
             jm8, rm16, ac16,
             jm_sh, rm_sh, ac_sh,
             jm, areap, rowmax, argcol, colmask, perm, tmpf, tmpi, outv):
    iota = lax.iota(jnp.int32, V)
    lane0 = iota == 0
    lane15 = iota == V - 1
    splat15 = jnp.full((V,), V - 1, jnp.int32)

    def splat_last(v):
        return jax.lax.gather(
            v, splat15[:, None],
            jax.lax.GatherDimensionNumbers(
                offset_dims=(), collapsed_slice_dims=(0,),
                start_index_map=(0,)),
            (1,), mode=jax.lax.GatherScatterMode.PROMISE_IN_BOUNDS)

    neg1f = jnp.full((V,), -1.0, jnp.float32)
    neg1i = jnp.full((V,), -1, jnp.int32)
    neg4f = jnp.full((V,), -4.0, jnp.float32)
    zerof = jnp.full((V,), 0.0, jnp.float32)

    sid = lax.axis_index("s")

    @pl.when(lax.axis_index("c") == 0)
    def _():
        pltpu.sync_copy(lab, labv)
        pltpu.sync_copy(pred, predv)
        for t, (lcol, pcol) in enumerate(
                ((lx1v, px1v), (ly1v, py1v), (lx2v, px2v), (ly2v, py2v))):
            for j in range(NCH):
                s = pl.ds(j * V, V)
                idx = (iota + j * V) * 4 + t
                lcol[s] = plsc.load_gather(labv, [idx])
                pcol[s] = plsc.load_gather(predv, [idx])
        for j in range(NCH):
            s = pl.ds(j * V, V)
            areap[s] = (px2v[s] - px1v[s]) * (py2v[s] - py1v[s])

        def max_lastarg(vms):
            lanemax = vms[0]
            for j in range(1, NCH):
                lanemax = jnp.maximum(lanemax, vms[j])
            lc = jnp.zeros((V,), jnp.int32)
            for j in range(1, NCH):
                lc = jnp.where(vms[j] == lanemax,
                               jnp.full((V,), j, jnp.int32), lc)
            cm = plsc.cummax(lanemax)
            mspl = splat_last(cm)
            flat = jnp.where(lanemax == mspl, lc * V + iota, neg1i)
            return cm, plsc.cummax(flat)

        for q in range(RPT):
            lb = _bi(sid * RPT + q)
            x1 = plsc.load_gather(lx1v, [lb])
            y1 = plsc.load_gather(ly1v, [lb])
            x2 = plsc.load_gather(lx2v, [lb])
            y2 = plsc.load_gather(ly2v, [lb])
            al = (x2 - x1) * (y2 - y1)
            vms = []
            for j in range(NCH):
                s = pl.ds(j * V, V)
                ix1 = jnp.maximum(x1, px1v[s])
                iy1 = jnp.maximum(y1, py1v[s])
                ix2 = jnp.minimum(x2, px2v[s])
                iy2 = jnp.minimum(y2, py2v[s])
                inter = jnp.maximum(ix2 - ix1, zerof) * jnp.maximum(
                    iy2 - iy1, zerof)
                v = inter / (al + areap[s] - inter)
                jm8[pl.ds(q * L + j * V, V)] = v
                vms.append(v)
            cm, cam = max_lastarg(vms)
            plsc.store_scatter(rm16, [_bi(q)], cm, mask=lane15)
            plsc.store_scatter(ac16, [_bi(q)], cam, mask=lane15)

        pltpu.sync_copy(jm8, jm_sh.at[pl.ds(sid * (RPT * L), RPT * L)])
        pltpu.sync_copy(rm16, rm_sh.at[pl.ds(sid * V, V)])
        pltpu.sync_copy(ac16, ac_sh.at[pl.ds(sid * V, V)])
        plsc.subcore_barrier()

        @pl.when(sid == 0)
        def _():
            pltpu.sync_copy(jm_sh, jm)
            pltpu.sync_copy(rm_sh, tmpf)
            pltpu.sync_copy(ac_sh, tmpi)
            for j in range(NCH):
                i = iota + j * V
                cidx = (i // RPT) * V + (i % RPT)
                rowmax[pl.ds(j * V, V)] = plsc.load_gather(tmpf, [cidx])
                argcol[pl.ds(j * V, V)] = plsc.load_gather(tmpi, [cidx])
                colmask[pl.ds(j * V, V)] = zerof
                perm[pl.ds(j * V, V)] = i

            def row_scan(riv, iv):
                base = riv * L
                vms = []
                for j in range(NCH):
                    v = plsc.load_gather(jm, [base + (iota + j * V)])
                    vms.append(v + colmask[pl.ds(j * V, V)])
                cm, cam = max_lastarg(vms)
                plsc.store_scatter(rowmax, [iv], cm, mask=lane15)
                plsc.store_scatter(argcol, [iv], cam, mask=lane15)
                return cm, cam

            lane01 = iota < 2
            lanevecs = [iota + j * V for j in range(NCH)]

            def step(k, carry):
                rms = list(carry[:NCH])
                acs = list(carry[NCH:])
                _, cam = max_lastarg(rms)
                rv = splat_last(cam)
                cv = plsc.load_gather(argcol, [rv])

                idx_rc = jnp.where(lane0, rv, cv)
                idx_cr = jnp.where(lane0, cv, rv)
                diag = rv == cv
                pswp = plsc.load_gather(perm, [idx_cr])
                bmax = plsc.load_gather(rowmax, [cv])
                bac = plsc.load_gather(argcol, [cv])
                plsc.store_scatter(perm, [idx_rc], pswp, mask=lane01)
                plsc.store_scatter(
                    rowmax, [idx_rc],
                    jnp.where(lane0 & ~diag, bmax, neg1f), mask=lane01)
                nac = jnp.where(lane0 & ~diag, bac, neg1i)
                plsc.store_scatter(argcol, [idx_rc], nac, mask=lane01)
                plsc.store_scatter(colmask, [cv], neg4f, mask=lane0)

                bacp = jnp.where(diag, neg1i, bac)
                bmaxp = jnp.where(diag, neg1f, bmax)
                cands = []
                hmax = neg1i
                for j in range(NCH):
                    lanevec = lanevecs[j]
                    rms[j] = jnp.where(lanevec == rv, bmaxp,
                                       jnp.where(lanevec == cv, neg1f,
                                                 rms[j]))
                    acs[j] = jnp.where(lanevec == rv, bacp,
                                       jnp.where(lanevec == cv, neg1i,
                                                 acs[j]))
                    cand = jnp.where(acs[j] == cv, lanevec, neg1i)
                    cands.append(cand)
                    hmax = jnp.maximum(hmax, cand)

                def rec_body(carry2):
                    i = carry2[0]
                    cnds = list(carry2[1:1 + NCH])
                    rms2 = list(carry2[1 + NCH:1 + 2 * NCH])
                    acs2 = list(carry2[1 + 2 * NCH:])
                    iv = _bi(i)
                    cm2, cam2 = row_scan(plsc.load_gather(perm, [iv]), iv)
                    m2spl = splat_last(cm2)
                    a2spl = splat_last(cam2)
                    nmax = neg1i
                    for j in range(NCH):
                        sel = lanevecs[j] == iv
                        rms2[j] = jnp.where(sel, m2spl, rms2[j])
                        acs2[j] = jnp.where(sel, a2spl, acs2[j])
                        cnds[j] = jnp.where(cnds[j] == iv, neg1i, cnds[j])
                        nmax = jnp.maximum(nmax, cnds[j])
                    return (jnp.max(nmax),) + tuple(cnds) + tuple(
                        rms2) + tuple(acs2)

                fin = lax.while_loop(
                    lambda cr: cr[0] >= 0, rec_body,
                    (jnp.max(hmax),) + tuple(cands) + tuple(rms) + tuple(acs))
                return fin[1 + NCH:]

            lax.fori_loop(
                0, L, step,
                tuple(rowmax[pl.ds(j * V, V)] for j in range(NCH))
                + tuple(argcol[pl.ds(j * V, V)] for j in range(NCH)))

            acc = zerof
            for j in range(NCH):
                s = pl.ds(j * V, V)
                idx = perm[s]
                for lv, pv in ((lx1v, px1v), (ly1v, py1v),
                               (lx2v, px2v), (ly2v, py2v)):
                    acc = acc + jnp.abs(plsc.load_gather(lv, [idx]) - pv[s])
            outv[...] = lax.broadcast_in_dim(
                jnp.sum(acc) * jnp.float32(1.0 / (4 * L)), (V,), ())
            pltpu.sync_copy(outv, out)


@jax.jit
def _run(cols):
    mesh = plsc.VectorSubcoreMesh(core_axis_name="c", subcore_axis_name="s")
    f = pl.kernel(
        _sc_body,
        out_type=jax.ShapeDtypeStruct((V,), jnp.float32),
        mesh=mesh,
        scratch_types=(
            [pltpu.VMEM((4 * L,), jnp.float32) for _ in range(2)]
            + [pltpu.VMEM((L,), jnp.float32) for _ in range(8)]
            + [pltpu.VMEM((RPT * L,), jnp.float32),
               pltpu.VMEM((V,), jnp.float32),
               pltpu.VMEM((V,), jnp.int32),
               pltpu.VMEM_SHARED((L * L,), jnp.float32),
               pltpu.VMEM_SHARED((V * V,), jnp.float32),
               pltpu.VMEM_SHARED((V * V,), jnp.int32),
               pltpu.VMEM((L * L,), jnp.float32),
               pltpu.VMEM((L,), jnp.float32),
               pltpu.VMEM((L,), jnp.float32),
               pltpu.VMEM((L,), jnp.int32),
               pltpu.VMEM((L,), jnp.float32),
               pltpu.VMEM((L,), jnp.int32),
               pltpu.VMEM((V * V,), jnp.float32),
               pltpu.VMEM((V * V,), jnp.int32),
               pltpu.VMEM((V,), jnp.float32)]),
        compiler_params=pltpu.CompilerParams(needs_layout_passes=False),
    )
    return f(*cols)[0]


def kernel(predictions, labels, scores):
    del scores
    return _run((labels.reshape(-1), predictions.reshape(-1)))


# speedup vs baseline: 33.5585x; 1.4990x over previous
import jax
import jax.numpy as jnp
from jax import lax
from jax.experimental import pallas as pl
from jax.experimental.pallas import tpu as pltpu
from jax.experimental.pallas import tpu_sc as plsc

L = 128
V = 16
NCH = L // V
RPT = L // V


def _bi(s):
    return lax.broadcast_in_dim(jnp.int32(s), (V,), ())


def _sc_body(lab, pred, out,
             labv, predv,
             lx1v, ly1v, lx2v, ly2v, px1v, py1v, px2v, py2v,
             jm8, rm16, ac16,
             jm_sh, rm_sh, ac_sh,
             jm, areap, rowmax, argcol, colmask, perm, tmpf, tmpi, outv):
    iota = lax.iota(jnp.int32, V)
    lane0 = iota == 0
    lane15 = iota == V - 1
    splat15 = jnp.full((V,), V - 1, jnp.int32)

    def splat_last(v):
        return jax.lax.gather(
            v, splat15[:, None],
            jax.lax.GatherDimensionNumbers(
                offset_dims=(), collapsed_slice_dims=(0,),
                start_index_map=(0,)),
            (1,), mode=jax.lax.GatherScatterMode.PROMISE_IN_BOUNDS)

    neg1f = jnp.full((V,), -1.0, jnp.float32)
    neg1i = jnp.full((V,), -1, jnp.int32)
    neg4f = jnp.full((V,), -4.0, jnp.float32)
    zerof = jnp.full((V,), 0.0, jnp.float32)
    zeroi = jnp.full((V,), 0, jnp.int32)

    sid = lax.axis_index("s")

    @pl.when(lax.axis_index("c") == 0)
    def _():
        pltpu.sync_copy(lab, labv)
        pltpu.sync_copy(pred, predv)
        for t, (lcol, pcol) in enumerate(
                ((lx1v, px1v), (ly1v, py1v), (lx2v, px2v), (ly2v, py2v))):
            for j in range(NCH):
                s = pl.ds(j * V, V)
                idx = (iota + j * V) * 4 + t
                lcol[s] = plsc.load_gather(labv, [idx])
                pcol[s] = plsc.load_gather(predv, [idx])
        for j in range(NCH):
            s = pl.ds(j * V, V)
            areap[s] = (px2v[s] - px1v[s]) * (py2v[s] - py1v[s])

        def max_lastarg(vms):
            lanemax = vms[0]
            for j in range(1, NCH):
                lanemax = jnp.maximum(lanemax, vms[j])
            lc = jnp.zeros((V,), jnp.int32)
            for j in range(1, NCH):
                lc = jnp.where(vms[j] == lanemax,
                               jnp.full((V,), j, jnp.int32), lc)
            cm = plsc.cummax(lanemax)
            mspl = splat_last(cm)
            flat = jnp.where((lanemax == mspl) & (mspl > zerof),
                             lc * V + iota, neg1i)
            return cm, plsc.cummax(flat)

        for q in range(RPT):
            lb = _bi(sid * RPT + q)
            x1 = plsc.load_gather(lx1v, [lb])
            y1 = plsc.load_gather(ly1v, [lb])
            x2 = plsc.load_gather(lx2v, [lb])
            y2 = plsc.load_gather(ly2v, [lb])
            al = (x2 - x1) * (y2 - y1)
            vms = []
            for j in range(NCH):
                s = pl.ds(j * V, V)
                ix1 = jnp.maximum(x1, px1v[s])
                iy1 = jnp.maximum(y1, py1v[s])
                ix2 = jnp.minimum(x2, px2v[s])
                iy2 = jnp.minimum(y2, py2v[s])
                inter = jnp.maximum(ix2 - ix1, zerof) * jnp.maximum(
                    iy2 - iy1, zerof)
                v = inter / (al + areap[s] - inter)
                jm8[pl.ds(q * L + j * V, V)] = v
                vms.append(v)
            cm, cam = max_lastarg(vms)
            plsc.store_scatter(rm16, [_bi(q)], cm, mask=lane15)
            plsc.store_scatter(ac16, [_bi(q)], cam, mask=lane15)

        pltpu.sync_copy(jm8, jm_sh.at[pl.ds(sid * (RPT * L), RPT * L)])
        pltpu.sync_copy(rm16, rm_sh.at[pl.ds(sid * V, V)])
        pltpu.sync_copy(ac16, ac_sh.at[pl.ds(sid * V, V)])
        plsc.subcore_barrier()

        @pl.when(sid == 0)
        def _():
            pltpu.sync_copy(jm_sh, jm)
            pltpu.sync_copy(rm_sh, tmpf)
            pltpu.sync_copy(ac_sh, tmpi)
            for j in range(NCH):
                i = iota + j * V
                cidx = (i // RPT) * V + (i % RPT)
                rowmax[pl.ds(j * V, V)] = plsc.load_gather(tmpf, [cidx])
                argcol[pl.ds(j * V, V)] = plsc.load_gather(tmpi, [cidx])
                colmask[pl.ds(j * V, V)] = zerof
                perm[pl.ds(j * V, V)] = i

            def row_scan(riv, iv):
                base = riv * L
                vms = []
                for j in range(NCH):
                    v = plsc.load_gather(jm, [base + (iota + j * V)])
                    vms.append(v + colmask[pl.ds(j * V, V)])
                cm, cam = max_lastarg(vms)
                plsc.store_scatter(rowmax, [iv], cm, mask=lane15)
                plsc.store_scatter(argcol, [iv], cam, mask=lane15)
                return cm, cam

            lane01 = iota < 2
            lanevecs = [iota + j * V for j in range(NCH)]

            def step(carry):
                k = carry[0]
                rms = list(carry[2:2 + NCH])
                acs = list(carry[2 + NCH:])
                _, cam = max_lastarg(rms)
                rv = splat_last(cam)
                ok = rv >= zeroi
                rvc = jnp.maximum(rv, zeroi)
                cv = plsc.load_gather(argcol, [rvc])
                cvc = jnp.maximum(cv, zeroi)

                idx_rc = jnp.where(lane0, rvc, cvc)
                idx_cr = jnp.where(lane0, cvc, rvc)
                diag = rv == cv
                swpm = lane01 & ok
                pswp = plsc.load_gather(perm, [idx_cr])
                bmax = plsc.load_gather(rowmax, [cvc])
                bac = plsc.load_gather(argcol, [cvc])
                plsc.store_scatter(perm, [idx_rc], pswp, mask=swpm)
                plsc.store_scatter(
                    rowmax, [idx_rc],
                    jnp.where(lane0 & ~diag, bmax, neg1f), mask=swpm)
                nac = jnp.where(lane0 & ~diag, bac, neg1i)
                plsc.store_scatter(argcol, [idx_rc], nac, mask=swpm)
                plsc.store_scatter(colmask, [cvc], neg4f, mask=lane0 & ok)

                bacp = jnp.where(diag, neg1i, bac)
                bmaxp = jnp.where(diag, neg1f, bmax)
                cands = []
                hmax = neg1i
                for j in range(NCH):
                    lanevec = lanevecs[j]
                    rms[j] = jnp.where(lanevec == rv, bmaxp,
                                       jnp.where(lanevec == cv, neg1f,
                                                 rms[j]))
                    acs[j] = jnp.where(lanevec == rv, bacp,
                                       jnp.where(lanevec == cv, neg1i,
                                                 acs[j]))
                    cand = jnp.where(acs[j] == cv, lanevec, neg1i)
                    cands.append(cand)
                    hmax = jnp.maximum(hmax, cand)
                hmax = jnp.where(ok, hmax, jnp.full((V,), -2, jnp.int32))

                def rec_body(carry2):
                    i = carry2[0]
                    cnds = list(carry2[1:1 + NCH])
                    rms2 = list(carry2[1 + NCH:1 + 2 * NCH])
                    acs2 = list(carry2[1 + 2 * NCH:])
                    iv = _bi(i)
                    cm2, cam2 = row_scan(plsc.load_gather(perm, [iv]), iv)
                    m2spl = splat_last(cm2)
                    a2spl = splat_last(cam2)
                    nmax = neg1i
                    for j in range(NCH):
                        sel = lanevecs[j] == iv
                        rms2[j] = jnp.where(sel, m2spl, rms2[j])
                        acs2[j] = jnp.where(sel, a2spl, acs2[j])
                        cnds[j] = jnp.where(cnds[j] == iv, neg1i, cnds[j])
                        nmax = jnp.maximum(nmax, cnds[j])
                    return (jnp.max(nmax),) + tuple(cnds) + tuple(
                        rms2) + tuple(acs2)

                i0 = jnp.max(hmax)
                fin = lax.while_loop(
                    lambda cr: cr[0] >= 0, rec_body,
                    (i0,) + tuple(cands) + tuple(rms) + tuple(acs))
                return (k + 1, jnp.int32(i0 > -2)) + fin[1 + NCH:]

            lax.while_loop(
                lambda cr: (cr[0] < L) & (cr[1] > 0), step,
                (jnp.int32(0), jnp.int32(1))
                + tuple(rowmax[pl.ds(j * V, V)] for j in range(NCH))
                + tuple(argcol[pl.ds(j * V, V)] for j in range(NCH)))

            acc = zerof
            for j in range(NCH):
                s = pl.ds(j * V, V)
                idx = perm[s]
                for lv, pv in ((lx1v, px1v), (ly1v, py1v),
                               (lx2v, px2v), (ly2v, py2v)):
                    acc = acc + jnp.abs(plsc.load_gather(lv, [idx]) - pv[s])
            outv[...] = lax.broadcast_in_dim(
                jnp.sum(acc) * jnp.float32(1.0 / (4 * L)), (V,), ())
            pltpu.sync_copy(outv, out)


@jax.jit
def _run(cols):
    mesh = plsc.VectorSubcoreMesh(core_axis_name="c", subcore_axis_name="s")
    f = pl.kernel(
        _sc_body,
        out_type=jax.ShapeDtypeStruct((V,), jnp.float32),
        mesh=mesh,
        scratch_types=(
            [pltpu.VMEM((4 * L,), jnp.float32) for _ in range(2)]
            + [pltpu.VMEM((L,), jnp.float32) for _ in range(8)]
            + [pltpu.VMEM((RPT * L,), jnp.float32),
               pltpu.VMEM((V,), jnp.float32),
               pltpu.VMEM((V,), jnp.int32),
               pltpu.VMEM_SHARED((L * L,), jnp.float32),
               pltpu.VMEM_SHARED((V * V,), jnp.float32),
               pltpu.VMEM_SHARED((V * V,), jnp.int32),
               pltpu.VMEM((L * L,), jnp.float32),
               pltpu.VMEM((L,), jnp.float32),
               pltpu.VMEM((L,), jnp.float32),
               pltpu.VMEM((L,), jnp.int32),
               pltpu.VMEM((L,), jnp.float32),
               pltpu.VMEM((L,), jnp.int32),
               pltpu.VMEM((V * V,), jnp.float32),
               pltpu.VMEM((V * V,), jnp.int32),
               pltpu.VMEM((V,), jnp.float32)]),
        compiler_params=pltpu.CompilerParams(needs_layout_passes=False),
    )
    return f(*cols)[0]


def kernel(predictions, labels, scores):
    del scores
    return _run((labels.reshape(-1), predictions.reshape(-1)))
